# Initial kernel scaffold; baseline (speedup 1.0000x reference)
import functools

import jax
import jax.numpy as jnp
from jax import lax
from jax.experimental import pallas as pl
from jax.experimental.pallas import tpu as pltpu
from jax.experimental.pallas import tpu_sc as plsc

N_NODES = 50000
N_EDGES = 800000
F_IN = 58

NC = 2
NS = 16
NW = NC * NS
L = 16

CHUNK = 128
NPAD = 50176
RPT = NPAD // NS
EPW = 25088
EPAD = EPW * NW
NCH = EPW // CHUNK

_sc_mesh = plsc.VectorSubcoreMesh(core_axis_name="c", subcore_axis_name="s")


def _worker_id():
    return lax.axis_index("c") * NS + lax.axis_index("s")


@functools.partial(
    pl.kernel,
    out_type=jax.ShapeDtypeStruct((NC, NPAD), jnp.float32),
    mesh=_sc_mesh,
    scratch_types=[
        pltpu.VMEM((NCH, CHUNK), jnp.int32),
        pltpu.VMEM((CHUNK,), jnp.float32),
        pltpu.VMEM_SHARED((NPAD,), jnp.float32),
    ],
)
def _sc_degree(col_hbm, zeros_hbm, out_hbm, cidx, ones_v, deg_sh):
    c = lax.axis_index("c")
    s = lax.axis_index("s")
    wid = _worker_id()
    pltpu.sync_copy(zeros_hbm.at[pl.ds(s * RPT, RPT)], deg_sh.at[pl.ds(s * RPT, RPT)])

    def onesb(i, _):
        ones_v[pl.ds(i * L, L)] = jnp.ones((L,), jnp.float32)
        return 0
    lax.fori_loop(0, CHUNK // L, onesb, 0)
    pltpu.sync_copy(col_hbm.at[wid], cidx)
    plsc.subcore_barrier()

    def body(j, _):
        pltpu.sync_copy(ones_v, deg_sh.at[cidx.at[j]], add=True)
        return 0
    lax.fori_loop(0, NCH, body, 0)
    plsc.subcore_barrier()
    pltpu.sync_copy(deg_sh.at[pl.ds(s * RPT, RPT)], out_hbm.at[c, pl.ds(s * RPT, RPT)])


@functools.partial(
    pl.kernel,
    out_type=jax.ShapeDtypeStruct((NC, NPAD, 32), jnp.float32),
    mesh=_sc_mesh,
    scratch_types=[
        pltpu.VMEM((NCH, CHUNK), jnp.int32),
        pltpu.VMEM((NCH, CHUNK), jnp.int32),
        pltpu.VMEM((CHUNK, 32), jnp.float32),
        pltpu.SemaphoreType.DMA,
        pltpu.VMEM_SHARED((NPAD, 32), jnp.float32),
    ],
)
def _sc_pass1(row_hbm, col_hbm, tab_hbm, zeros_hbm, out_hbm,
              ridx, cidx, data, sem, acc_sh):
    c = lax.axis_index("c")
    s = lax.axis_index("s")
    wid = _worker_id()
    pltpu.sync_copy(zeros_hbm.at[pl.ds(s * RPT, RPT)], acc_sh.at[pl.ds(s * RPT, RPT)])
    pltpu.sync_copy(row_hbm.at[wid], ridx)
    pltpu.sync_copy(col_hbm.at[wid], cidx)
    plsc.subcore_barrier()

    def body(j, _):
        pltpu.async_copy(tab_hbm.at[ridx.at[j]], data, sem).wait()
        pltpu.sync_copy(data, acc_sh.at[cidx.at[j]], add=True)
        return 0
    lax.fori_loop(0, NCH, body, 0)
    plsc.subcore_barrier()
    pltpu.sync_copy(acc_sh.at[pl.ds(s * RPT, RPT)],
                    out_hbm.at[c, pl.ds(s * RPT, RPT), :])


@functools.partial(
    pl.kernel,
    out_type=[jax.ShapeDtypeStruct((NC, NPAD), jnp.float32),
              jax.ShapeDtypeStruct((NC, NPAD), jnp.float32)],
    mesh=_sc_mesh,
    scratch_types=[
        pltpu.VMEM((NCH, CHUNK), jnp.int32),
        pltpu.VMEM((NCH, CHUNK), jnp.int32),
        pltpu.VMEM((CHUNK,), jnp.float32),
        pltpu.VMEM((CHUNK,), jnp.float32),
        pltpu.SemaphoreType.DMA,
        pltpu.SemaphoreType.DMA,
        pltpu.VMEM_SHARED((NPAD,), jnp.float32),
        pltpu.VMEM_SHARED((NPAD,), jnp.float32),
    ],
)
def _sc_pass2(row_hbm, col_hbm, z0_hbm, z1_hbm, zeros_hbm, out0_hbm, out1_hbm,
              ridx, cidx, d0, d1, sem0, sem1, g0_sh, g1_sh):
    c = lax.axis_index("c")
    s = lax.axis_index("s")
    wid = _worker_id()
    pltpu.sync_copy(zeros_hbm.at[pl.ds(s * RPT, RPT)], g0_sh.at[pl.ds(s * RPT, RPT)])
    pltpu.sync_copy(zeros_hbm.at[pl.ds(s * RPT, RPT)], g1_sh.at[pl.ds(s * RPT, RPT)])
    pltpu.sync_copy(row_hbm.at[wid], ridx)
    pltpu.sync_copy(col_hbm.at[wid], cidx)
    plsc.subcore_barrier()

    def body(j, _):
        cp0 = pltpu.async_copy(z0_hbm.at[ridx.at[j]], d0, sem0)
        cp1 = pltpu.async_copy(z1_hbm.at[ridx.at[j]], d1, sem1)
        cp0.wait()
        cp1.wait()
        pltpu.sync_copy(d0, g0_sh.at[cidx.at[j]], add=True)
        pltpu.sync_copy(d1, g1_sh.at[cidx.at[j]], add=True)
        return 0
    lax.fori_loop(0, NCH, body, 0)
    plsc.subcore_barrier()
    pltpu.sync_copy(g0_sh.at[pl.ds(s * RPT, RPT)], out0_hbm.at[c, pl.ds(s * RPT, RPT)])
    pltpu.sync_copy(g1_sh.at[pl.ds(s * RPT, RPT)], out1_hbm.at[c, pl.ds(s * RPT, RPT)])


_BLK = RPT
_GRID = NPAD // _BLK


def _tc_a_body(x_ref, w_ref, o_ref):
    o_ref[...] = jnp.dot(x_ref[...], w_ref[...],
                         preferred_element_type=jnp.float32)


def _tc_a(x_p, w_cat):
    return pl.pallas_call(
        _tc_a_body,
        grid=(_GRID,),
        in_specs=[pl.BlockSpec((_BLK, F_IN), lambda i: (i, 0)),
                  pl.BlockSpec((F_IN, 64), lambda i: (0, 0))],
        out_specs=pl.BlockSpec((_BLK, 64), lambda i: (i, 0)),
        out_shape=jax.ShapeDtypeStruct((NPAD, 64), jnp.float32),
    )(x_p, w_cat)


def _dinv_from(degp_ref):
    deg = degp_ref[0, :] + degp_ref[1, :]
    return jnp.where(deg > 0, lax.rsqrt(jnp.maximum(deg, 1e-12)), 0.0)


def _tc_b_body(p_ref, degp_ref, hs_ref):
    dinv = _dinv_from(degp_ref)
    hs_ref[...] = p_ref[:, :32] * dinv[:, None]


def _tc_b(p, deg_part):
    return pl.pallas_call(
        _tc_b_body,
        grid=(_GRID,),
        in_specs=[pl.BlockSpec((_BLK, 64), lambda i: (i, 0)),
                  pl.BlockSpec((NC, _BLK), lambda i: (0, i))],
        out_specs=pl.BlockSpec((_BLK, 32), lambda i: (i, 0)),
        out_shape=jax.ShapeDtypeStruct((NPAD, 32), jnp.float32),
    )(p, deg_part)


def _tc_c_body(gp_ref, p_ref, degp_ref, b1_ref, w2_ref,
               z0_ref, z1_ref, r2_ref):
    dinv = _dinv_from(degp_ref)
    g = gp_ref[0] + gp_ref[1]
    agg = g * dinv[:, None]
    pre0 = agg[:, :16] + p_ref[:, 32:48] + b1_ref[0, :][None, :]
    pre1 = agg[:, 16:32] + p_ref[:, 48:64] + b1_ref[1, :][None, :]
    h = jax.nn.relu(0.5 * (jax.nn.relu(pre0) + jax.nn.relu(pre1)))
    z = jnp.dot(h, w2_ref[...], preferred_element_type=jnp.float32)
    z0_ref[...] = z[:, 0:1] * dinv[:, None]
    z1_ref[...] = z[:, 1:2] * dinv[:, None]
    r2_ref[...] = z[:, 2:4]


def _tc_c(g_part, p, deg_part, b1, w2_cat):
    return pl.pallas_call(
        _tc_c_body,
        grid=(_GRID,),
        in_specs=[pl.BlockSpec((NC, _BLK, 32), lambda i: (0, i, 0)),
                  pl.BlockSpec((_BLK, 64), lambda i: (i, 0)),
                  pl.BlockSpec((NC, _BLK), lambda i: (0, i)),
                  pl.BlockSpec((2, 16), lambda i: (0, 0)),
                  pl.BlockSpec((16, 4), lambda i: (0, 0))],
        out_specs=[pl.BlockSpec((_BLK, 1), lambda i: (i, 0)),
                   pl.BlockSpec((_BLK, 1), lambda i: (i, 0)),
                   pl.BlockSpec((_BLK, 2), lambda i: (i, 0))],
        out_shape=[jax.ShapeDtypeStruct((NPAD, 1), jnp.float32),
                   jax.ShapeDtypeStruct((NPAD, 1), jnp.float32),
                   jax.ShapeDtypeStruct((NPAD, 2), jnp.float32)],
    )(g_part, p, deg_part, b1, w2_cat)


def _tc_d_body(g0_ref, g1_ref, degp_ref, r2_ref, b2_ref, o_ref):
    dinv = _dinv_from(degp_ref)
    o0 = jax.nn.relu(dinv * (g0_ref[0] + g0_ref[1]) + r2_ref[:, 0] + b2_ref[0, 0])
    o1 = jax.nn.relu(dinv * (g1_ref[0] + g1_ref[1]) + r2_ref[:, 1] + b2_ref[1, 0])
    o_ref[...] = (0.5 * (o0 + o1))[:, None]


def _tc_d(g2p0, g2p1, deg_part, r2, b2):
    return pl.pallas_call(
        _tc_d_body,
        grid=(_GRID,),
        in_specs=[pl.BlockSpec((NC, _BLK), lambda i: (0, i)),
                  pl.BlockSpec((NC, _BLK), lambda i: (0, i)),
                  pl.BlockSpec((NC, _BLK), lambda i: (0, i)),
                  pl.BlockSpec((_BLK, 2), lambda i: (i, 0)),
                  pl.BlockSpec((2, 1), lambda i: (0, 0))],
        out_specs=pl.BlockSpec((_BLK, 1), lambda i: (i, 0)),
        out_shape=jax.ShapeDtypeStruct((NPAD, 1), jnp.float32),
    )(g2p0, g2p1, deg_part, r2, b2)


def kernel(x, edge_index, W1_init, W1_root, b1, W2_init, W2_root, b2):
    f32 = jnp.float32
    row = edge_index[0].astype(jnp.int32)
    col = edge_index[1].astype(jnp.int32)
    pad_e = jnp.full((EPAD - N_EDGES,), N_NODES, jnp.int32)
    row_r = jnp.concatenate([row, pad_e]).reshape(NW, NCH, CHUNK)
    col_r = jnp.concatenate([col, pad_e]).reshape(NW, NCH, CHUNK)

    x_p = jnp.pad(x.astype(f32), ((0, NPAD - N_NODES), (0, 0)))
    w1_cat = jnp.concatenate(
        [W1_init[0], W1_init[1], W1_root[0], W1_root[1]], axis=1).astype(f32)
    w2_cat = jnp.concatenate(
        [W2_init[0], W2_init[1], W2_root[0], W2_root[1]], axis=1).astype(f32)

    zeros_n = jnp.zeros((NPAD,), f32)
    zeros_n32 = jnp.zeros((NPAD, 32), f32)

    p = _tc_a(x_p, w1_cat)
    deg_part = _sc_degree(col_r, zeros_n)
    hs = _tc_b(p, deg_part)
    g_part = _sc_pass1(row_r, col_r, hs, zeros_n32)
    z0, z1, r2 = _tc_c(g_part, p, deg_part, b1.astype(f32), w2_cat)
    z0 = z0.reshape(NPAD)
    z1 = z1.reshape(NPAD)
    g2p0, g2p1 = _sc_pass2(row_r, col_r, z0, z1, zeros_n)
    out = _tc_d(g2p0, g2p1, deg_part, r2, b2.astype(f32))
    return out[:N_NODES]


# speedup vs baseline: 105.7351x; 105.7351x over previous
import functools

import jax
import jax.numpy as jnp
from jax import lax
from jax.experimental import pallas as pl
from jax.experimental.pallas import tpu as pltpu
from jax.experimental.pallas import tpu_sc as plsc

N_NODES = 50000
N_EDGES = 800000
F_IN = 58

NC = 2
NS = 16
NW = NC * NS
L = 16

CHUNK = 128
NPAD = 50176
RPT = NPAD // NS
EPW = 25088
EPAD = EPW * NW
NCH = EPW // CHUNK
GSZ = 28
NGRP = NCH // GSZ

_sc_mesh = plsc.VectorSubcoreMesh(core_axis_name="c", subcore_axis_name="s")


def _worker_id():
    return lax.axis_index("c") * NS + lax.axis_index("s")


@functools.partial(
    pl.kernel,
    out_type=jax.ShapeDtypeStruct((NC, NPAD), jnp.float32),
    mesh=_sc_mesh,
    compiler_params=pltpu.CompilerParams(use_tc_tiling_on_sc=False),
    scratch_types=[
        pltpu.VMEM((NCH, CHUNK), jnp.int32),
        pltpu.VMEM((CHUNK,), jnp.float32),
        pltpu.VMEM_SHARED((NPAD,), jnp.float32),
    ],
)
def _sc_degree(col_hbm, zeros_hbm, out_hbm, cidx, ones_v, deg_sh):
    c = lax.axis_index("c")
    s = lax.axis_index("s")
    wid = _worker_id()
    pltpu.sync_copy(zeros_hbm.at[pl.ds(s * RPT, RPT)], deg_sh.at[pl.ds(s * RPT, RPT)])

    def onesb(i, _):
        ones_v[pl.ds(i * L, L)] = jnp.ones((L,), jnp.float32)
        return 0
    lax.fori_loop(0, CHUNK // L, onesb, 0)
    pltpu.sync_copy(col_hbm.at[wid], cidx)
    plsc.subcore_barrier()

    def body(j, _):
        pltpu.sync_copy(ones_v, deg_sh.at[cidx.at[j]], add=True)
        return 0
    lax.fori_loop(0, NCH, body, 0)
    plsc.subcore_barrier()
    pltpu.sync_copy(deg_sh.at[pl.ds(s * RPT, RPT)], out_hbm.at[c, pl.ds(s * RPT, RPT)])


@functools.partial(
    pl.kernel,
    out_type=jax.ShapeDtypeStruct((NC, NPAD, 32), jnp.float32),
    mesh=_sc_mesh,
    compiler_params=pltpu.CompilerParams(use_tc_tiling_on_sc=False),
    scratch_types=[
        pltpu.VMEM((GSZ, CHUNK), jnp.int32),
        pltpu.VMEM((GSZ, CHUNK), jnp.int32),
        pltpu.VMEM((CHUNK, 32), jnp.float32),
        pltpu.SemaphoreType.DMA,
        pltpu.VMEM_SHARED((NPAD, 32), jnp.float32),
    ],
)
def _sc_pass1(row_hbm, col_hbm, tab_hbm, zeros_hbm, out_hbm,
              ridx, cidx, data, sem, acc_sh):
    c = lax.axis_index("c")
    s = lax.axis_index("s")
    wid = _worker_id()
    pltpu.sync_copy(zeros_hbm.at[pl.ds(s * RPT, RPT)], acc_sh.at[pl.ds(s * RPT, RPT)])
    plsc.subcore_barrier()

    def grp(gi, _):
        pltpu.sync_copy(row_hbm.at[wid, pl.ds(gi * GSZ, GSZ)], ridx)
        pltpu.sync_copy(col_hbm.at[wid, pl.ds(gi * GSZ, GSZ)], cidx)

        def body(j, _):
            pltpu.async_copy(tab_hbm.at[ridx.at[j]], data, sem).wait()
            pltpu.sync_copy(data, acc_sh.at[cidx.at[j]], add=True)
            return 0
        lax.fori_loop(0, GSZ, body, 0)
        return 0
    lax.fori_loop(0, NGRP, grp, 0)
    plsc.subcore_barrier()
    pltpu.sync_copy(acc_sh.at[pl.ds(s * RPT, RPT)],
                    out_hbm.at[c, pl.ds(s * RPT, RPT), :])


@functools.partial(
    pl.kernel,
    out_type=[jax.ShapeDtypeStruct((NC, NPAD), jnp.float32),
              jax.ShapeDtypeStruct((NC, NPAD), jnp.float32)],
    mesh=_sc_mesh,
    compiler_params=pltpu.CompilerParams(use_tc_tiling_on_sc=False),
    scratch_types=[
        pltpu.VMEM((NCH, CHUNK), jnp.int32),
        pltpu.VMEM((NCH, CHUNK), jnp.int32),
        pltpu.VMEM((CHUNK,), jnp.float32),
        pltpu.VMEM((CHUNK,), jnp.float32),
        pltpu.SemaphoreType.DMA,
        pltpu.SemaphoreType.DMA,
        pltpu.VMEM_SHARED((NPAD,), jnp.float32),
        pltpu.VMEM_SHARED((NPAD,), jnp.float32),
    ],
)
def _sc_pass2(row_hbm, col_hbm, z0_hbm, z1_hbm, zeros_hbm, out0_hbm, out1_hbm,
              ridx, cidx, d0, d1, sem0, sem1, g0_sh, g1_sh):
    c = lax.axis_index("c")
    s = lax.axis_index("s")
    wid = _worker_id()
    pltpu.sync_copy(zeros_hbm.at[pl.ds(s * RPT, RPT)], g0_sh.at[pl.ds(s * RPT, RPT)])
    pltpu.sync_copy(zeros_hbm.at[pl.ds(s * RPT, RPT)], g1_sh.at[pl.ds(s * RPT, RPT)])
    pltpu.sync_copy(row_hbm.at[wid], ridx)
    pltpu.sync_copy(col_hbm.at[wid], cidx)
    plsc.subcore_barrier()

    def body(j, _):
        cp0 = pltpu.async_copy(z0_hbm.at[ridx.at[j]], d0, sem0)
        cp1 = pltpu.async_copy(z1_hbm.at[ridx.at[j]], d1, sem1)
        cp0.wait()
        cp1.wait()
        pltpu.sync_copy(d0, g0_sh.at[cidx.at[j]], add=True)
        pltpu.sync_copy(d1, g1_sh.at[cidx.at[j]], add=True)
        return 0
    lax.fori_loop(0, NCH, body, 0)
    plsc.subcore_barrier()
    pltpu.sync_copy(g0_sh.at[pl.ds(s * RPT, RPT)], out0_hbm.at[c, pl.ds(s * RPT, RPT)])
    pltpu.sync_copy(g1_sh.at[pl.ds(s * RPT, RPT)], out1_hbm.at[c, pl.ds(s * RPT, RPT)])


_BLK = RPT
_GRID = NPAD // _BLK


def _tc_a_body(x_ref, w_ref, o_ref):
    o_ref[...] = jnp.dot(x_ref[...], w_ref[...],
                         preferred_element_type=jnp.float32)


def _tc_a(x_p, w_cat):
    return pl.pallas_call(
        _tc_a_body,
        grid=(_GRID,),
        in_specs=[pl.BlockSpec((_BLK, F_IN), lambda i: (i, 0)),
                  pl.BlockSpec((F_IN, 64), lambda i: (0, 0))],
        out_specs=pl.BlockSpec((_BLK, 64), lambda i: (i, 0)),
        out_shape=jax.ShapeDtypeStruct((NPAD, 64), jnp.float32),
    )(x_p, w_cat)


def _dinv_from(degp_ref):
    deg = degp_ref[0, :, 0] + degp_ref[1, :, 0]
    return jnp.where(deg > 0, lax.rsqrt(jnp.maximum(deg, 1e-12)), 0.0)


def _tc_b_body(p_ref, degp_ref, hs_ref):
    dinv = _dinv_from(degp_ref)
    hs_ref[...] = p_ref[:, :32] * dinv[:, None]


def _tc_b(p, deg_part):
    return pl.pallas_call(
        _tc_b_body,
        grid=(_GRID,),
        in_specs=[pl.BlockSpec((_BLK, 64), lambda i: (i, 0)),
                  pl.BlockSpec((NC, _BLK, 1), lambda i: (0, i, 0))],
        out_specs=pl.BlockSpec((_BLK, 32), lambda i: (i, 0)),
        out_shape=jax.ShapeDtypeStruct((NPAD, 32), jnp.float32),
    )(p, deg_part)


def _tc_c_body(gp_ref, p_ref, degp_ref, b1_ref, w2_ref,
               z0_ref, z1_ref, r2_ref):
    dinv = _dinv_from(degp_ref)
    g = gp_ref[0] + gp_ref[1]
    agg = g * dinv[:, None]
    pre0 = agg[:, :16] + p_ref[:, 32:48] + b1_ref[0, :][None, :]
    pre1 = agg[:, 16:32] + p_ref[:, 48:64] + b1_ref[1, :][None, :]
    h = jax.nn.relu(0.5 * (jax.nn.relu(pre0) + jax.nn.relu(pre1)))
    z = jnp.dot(h, w2_ref[...], preferred_element_type=jnp.float32)
    z0_ref[...] = z[:, 0:1] * dinv[:, None]
    z1_ref[...] = z[:, 1:2] * dinv[:, None]
    r2_ref[...] = z[:, 2:4]


def _tc_c(g_part, p, deg_part, b1, w2_cat):
    return pl.pallas_call(
        _tc_c_body,
        grid=(_GRID,),
        in_specs=[pl.BlockSpec((NC, _BLK, 32), lambda i: (0, i, 0)),
                  pl.BlockSpec((_BLK, 64), lambda i: (i, 0)),
                  pl.BlockSpec((NC, _BLK, 1), lambda i: (0, i, 0)),
                  pl.BlockSpec((2, 16), lambda i: (0, 0)),
                  pl.BlockSpec((16, 4), lambda i: (0, 0))],
        out_specs=[pl.BlockSpec((_BLK, 1), lambda i: (i, 0)),
                   pl.BlockSpec((_BLK, 1), lambda i: (i, 0)),
                   pl.BlockSpec((_BLK, 2), lambda i: (i, 0))],
        out_shape=[jax.ShapeDtypeStruct((NPAD, 1), jnp.float32),
                   jax.ShapeDtypeStruct((NPAD, 1), jnp.float32),
                   jax.ShapeDtypeStruct((NPAD, 2), jnp.float32)],
    )(g_part, p, deg_part, b1, w2_cat)


def _tc_d_body(g0_ref, g1_ref, degp_ref, r2_ref, b2_ref, o_ref):
    dinv = _dinv_from(degp_ref)
    o0 = jax.nn.relu(dinv * (g0_ref[0, :, 0] + g0_ref[1, :, 0])
                     + r2_ref[:, 0] + b2_ref[0, 0])
    o1 = jax.nn.relu(dinv * (g1_ref[0, :, 0] + g1_ref[1, :, 0])
                     + r2_ref[:, 1] + b2_ref[1, 0])
    o_ref[...] = (0.5 * (o0 + o1))[:, None]


def _tc_d(g2p0, g2p1, deg_part, r2, b2):
    return pl.pallas_call(
        _tc_d_body,
        grid=(_GRID,),
        in_specs=[pl.BlockSpec((NC, _BLK, 1), lambda i: (0, i, 0)),
                  pl.BlockSpec((NC, _BLK, 1), lambda i: (0, i, 0)),
                  pl.BlockSpec((NC, _BLK, 1), lambda i: (0, i, 0)),
                  pl.BlockSpec((_BLK, 2), lambda i: (i, 0)),
                  pl.BlockSpec((2, 1), lambda i: (0, 0))],
        out_specs=pl.BlockSpec((_BLK, 1), lambda i: (i, 0)),
        out_shape=jax.ShapeDtypeStruct((NPAD, 1), jnp.float32),
    )(g2p0, g2p1, deg_part, r2, b2)


def kernel(x, edge_index, W1_init, W1_root, b1, W2_init, W2_root, b2):
    f32 = jnp.float32
    row = edge_index[0].astype(jnp.int32)
    col = edge_index[1].astype(jnp.int32)
    pad_e = jnp.full((EPAD - N_EDGES,), N_NODES, jnp.int32)
    row_r = jnp.concatenate([row, pad_e]).reshape(NW, NCH, CHUNK)
    col_r = jnp.concatenate([col, pad_e]).reshape(NW, NCH, CHUNK)

    x_p = jnp.pad(x.astype(f32), ((0, NPAD - N_NODES), (0, 0)))
    w1_cat = jnp.concatenate(
        [W1_init[0], W1_init[1], W1_root[0], W1_root[1]], axis=1).astype(f32)
    w2_cat = jnp.concatenate(
        [W2_init[0], W2_init[1], W2_root[0], W2_root[1]], axis=1).astype(f32)

    zeros_n = jnp.zeros((NPAD,), f32)
    zeros_n32 = jnp.zeros((NPAD, 32), f32)

    p = _tc_a(x_p, w1_cat)
    deg_part = _sc_degree(col_r, zeros_n)
    deg3 = deg_part.reshape(NC, NPAD, 1)
    hs = _tc_b(p, deg3)
    g_part = _sc_pass1(row_r, col_r, hs, zeros_n32)
    z0, z1, r2 = _tc_c(g_part, p, deg3, b1.astype(f32), w2_cat)
    z0 = z0.reshape(NPAD)
    z1 = z1.reshape(NPAD)
    g2p0, g2p1 = _sc_pass2(row_r, col_r, z0, z1, zeros_n)
    out = _tc_d(g2p0.reshape(NC, NPAD, 1), g2p1.reshape(NC, NPAD, 1),
                deg3, r2, b2.astype(f32))
    return out[:N_NODES]


# speedup vs baseline: 121.7207x; 1.1512x over previous
import functools

import jax
import jax.numpy as jnp
from jax import lax
from jax.experimental import pallas as pl
from jax.experimental.pallas import tpu as pltpu
from jax.experimental.pallas import tpu_sc as plsc

N_NODES = 50000
N_EDGES = 800000
F_IN = 58

NC = 2
NS = 16
NW = NC * NS
L = 16

CHUNK = 128
NPAD = 50176
RPT = NPAD // NS
EPW = 25088
EPAD = EPW * NW
NCH = EPW // CHUNK
GSZ = 28
ZW = 8
NGRP = NCH // GSZ

_sc_mesh = plsc.VectorSubcoreMesh(core_axis_name="c", subcore_axis_name="s")


def _worker_id():
    return lax.axis_index("c") * NS + lax.axis_index("s")


@functools.partial(
    pl.kernel,
    out_type=jax.ShapeDtypeStruct((NC, NPAD), jnp.float32),
    mesh=_sc_mesh,
    compiler_params=pltpu.CompilerParams(use_tc_tiling_on_sc=False),
    scratch_types=[
        pltpu.VMEM((NCH, CHUNK), jnp.int32),
        pltpu.VMEM((CHUNK,), jnp.float32),
        pltpu.VMEM_SHARED((NPAD,), jnp.float32),
    ],
)
def _sc_degree(col_hbm, zeros_hbm, out_hbm, cidx, ones_v, deg_sh):
    c = lax.axis_index("c")
    s = lax.axis_index("s")
    wid = _worker_id()
    pltpu.sync_copy(zeros_hbm.at[pl.ds(s * RPT, RPT)], deg_sh.at[pl.ds(s * RPT, RPT)])

    def onesb(i, _):
        ones_v[pl.ds(i * L, L)] = jnp.ones((L,), jnp.float32)
        return 0
    lax.fori_loop(0, CHUNK // L, onesb, 0)
    pltpu.sync_copy(col_hbm.at[wid], cidx)
    plsc.subcore_barrier()

    def body(j, _):
        pltpu.sync_copy(ones_v, deg_sh.at[cidx.at[j]], add=True)
        return 0
    lax.fori_loop(0, NCH, body, 0)
    plsc.subcore_barrier()
    pltpu.sync_copy(deg_sh.at[pl.ds(s * RPT, RPT)], out_hbm.at[c, pl.ds(s * RPT, RPT)])


@functools.partial(
    pl.kernel,
    out_type=jax.ShapeDtypeStruct((NC, NPAD, 32), jnp.float32),
    mesh=_sc_mesh,
    compiler_params=pltpu.CompilerParams(use_tc_tiling_on_sc=False),
    scratch_types=[
        pltpu.VMEM((GSZ, CHUNK), jnp.int32),
        pltpu.VMEM((GSZ, CHUNK), jnp.int32),
        pltpu.VMEM((CHUNK, 32), jnp.float32),
        pltpu.SemaphoreType.DMA,
        pltpu.VMEM_SHARED((NPAD, 32), jnp.float32),
    ],
)
def _sc_pass1(row_hbm, col_hbm, tab_hbm, zeros_hbm, out_hbm,
              ridx, cidx, data, sem, acc_sh):
    c = lax.axis_index("c")
    s = lax.axis_index("s")
    wid = _worker_id()
    pltpu.sync_copy(zeros_hbm.at[pl.ds(s * RPT, RPT)], acc_sh.at[pl.ds(s * RPT, RPT)])
    plsc.subcore_barrier()

    def grp(gi, _):
        pltpu.sync_copy(row_hbm.at[wid, pl.ds(gi * GSZ, GSZ)], ridx)
        pltpu.sync_copy(col_hbm.at[wid, pl.ds(gi * GSZ, GSZ)], cidx)

        def body(j, _):
            pltpu.async_copy(tab_hbm.at[ridx.at[j]], data, sem).wait()
            pltpu.sync_copy(data, acc_sh.at[cidx.at[j]], add=True)
            return 0
        lax.fori_loop(0, GSZ, body, 0)
        return 0
    lax.fori_loop(0, NGRP, grp, 0)
    plsc.subcore_barrier()
    pltpu.sync_copy(acc_sh.at[pl.ds(s * RPT, RPT)],
                    out_hbm.at[c, pl.ds(s * RPT, RPT), :])


@functools.partial(
    pl.kernel,
    out_type=jax.ShapeDtypeStruct((NC, NPAD, ZW), jnp.float32),
    mesh=_sc_mesh,
    compiler_params=pltpu.CompilerParams(use_tc_tiling_on_sc=False),
    scratch_types=[
        pltpu.VMEM((NCH, CHUNK), jnp.int32),
        pltpu.VMEM((NCH, CHUNK), jnp.int32),
        pltpu.VMEM((CHUNK, ZW), jnp.float32),
        pltpu.SemaphoreType.DMA,
        pltpu.VMEM_SHARED((NPAD, ZW), jnp.float32),
    ],
)
def _sc_pass2(row_hbm, col_hbm, z_hbm, zeros_hbm, out_hbm,
              ridx, cidx, d, sem, g_sh):
    c = lax.axis_index("c")
    s = lax.axis_index("s")
    wid = _worker_id()
    pltpu.sync_copy(zeros_hbm.at[pl.ds(s * RPT, RPT)], g_sh.at[pl.ds(s * RPT, RPT)])
    pltpu.sync_copy(row_hbm.at[wid], ridx)
    pltpu.sync_copy(col_hbm.at[wid], cidx)
    plsc.subcore_barrier()

    def body(j, _):
        pltpu.async_copy(z_hbm.at[ridx.at[j]], d, sem).wait()
        pltpu.sync_copy(d, g_sh.at[cidx.at[j]], add=True)
        return 0
    lax.fori_loop(0, NCH, body, 0)
    plsc.subcore_barrier()
    pltpu.sync_copy(g_sh.at[pl.ds(s * RPT, RPT)],
                    out_hbm.at[c, pl.ds(s * RPT, RPT), :])


_BLK = RPT
_GRID = NPAD // _BLK


def _tc_a_body(x_ref, w_ref, o_ref):
    o_ref[...] = jnp.dot(x_ref[...], w_ref[...],
                         preferred_element_type=jnp.float32)


def _tc_a(x_p, w_cat):
    return pl.pallas_call(
        _tc_a_body,
        grid=(_GRID,),
        in_specs=[pl.BlockSpec((_BLK, F_IN), lambda i: (i, 0)),
                  pl.BlockSpec((F_IN, 64), lambda i: (0, 0))],
        out_specs=pl.BlockSpec((_BLK, 64), lambda i: (i, 0)),
        out_shape=jax.ShapeDtypeStruct((NPAD, 64), jnp.float32),
    )(x_p, w_cat)


def _dinv_from(degp_ref):
    deg = degp_ref[0, :, 0] + degp_ref[1, :, 0]
    return jnp.where(deg > 0, lax.rsqrt(jnp.maximum(deg, 1e-12)), 0.0)


def _tc_b_body(p_ref, degp_ref, hs_ref):
    dinv = _dinv_from(degp_ref)
    hs_ref[...] = p_ref[:, :32] * dinv[:, None]


def _tc_b(p, deg_part):
    return pl.pallas_call(
        _tc_b_body,
        grid=(_GRID,),
        in_specs=[pl.BlockSpec((_BLK, 64), lambda i: (i, 0)),
                  pl.BlockSpec((NC, _BLK, 1), lambda i: (0, i, 0))],
        out_specs=pl.BlockSpec((_BLK, 32), lambda i: (i, 0)),
        out_shape=jax.ShapeDtypeStruct((NPAD, 32), jnp.float32),
    )(p, deg_part)


def _tc_c_body(gp_ref, p_ref, degp_ref, b1_ref, w2_ref,
               z_ref, r2_ref):
    dinv = _dinv_from(degp_ref)
    g = gp_ref[0] + gp_ref[1]
    agg = g * dinv[:, None]
    pre0 = agg[:, :16] + p_ref[:, 32:48] + b1_ref[0, :][None, :]
    pre1 = agg[:, 16:32] + p_ref[:, 48:64] + b1_ref[1, :][None, :]
    h = jax.nn.relu(0.5 * (jax.nn.relu(pre0) + jax.nn.relu(pre1)))
    z = jnp.dot(h, w2_ref[...], preferred_element_type=jnp.float32)
    zs = z[:, 0:2] * dinv[:, None]
    z_ref[...] = jnp.pad(zs, ((0, 0), (0, ZW - 2)))
    r2_ref[...] = z[:, 2:4]


def _tc_c(g_part, p, deg_part, b1, w2_cat):
    return pl.pallas_call(
        _tc_c_body,
        grid=(_GRID,),
        in_specs=[pl.BlockSpec((NC, _BLK, 32), lambda i: (0, i, 0)),
                  pl.BlockSpec((_BLK, 64), lambda i: (i, 0)),
                  pl.BlockSpec((NC, _BLK, 1), lambda i: (0, i, 0)),
                  pl.BlockSpec((2, 16), lambda i: (0, 0)),
                  pl.BlockSpec((16, 4), lambda i: (0, 0))],
        out_specs=[pl.BlockSpec((_BLK, ZW), lambda i: (i, 0)),
                   pl.BlockSpec((_BLK, 2), lambda i: (i, 0))],
        out_shape=[jax.ShapeDtypeStruct((NPAD, ZW), jnp.float32),
                   jax.ShapeDtypeStruct((NPAD, 2), jnp.float32)],
    )(g_part, p, deg_part, b1, w2_cat)


def _tc_d_body(g_ref, degp_ref, r2_ref, b2_ref, o_ref):
    dinv = _dinv_from(degp_ref)
    gsum = g_ref[0] + g_ref[1]
    o0 = jax.nn.relu(dinv * gsum[:, 0] + r2_ref[:, 0] + b2_ref[0, 0])
    o1 = jax.nn.relu(dinv * gsum[:, 1] + r2_ref[:, 1] + b2_ref[1, 0])
    o_ref[...] = (0.5 * (o0 + o1))[:, None]


def _tc_d(g2p, deg_part, r2, b2):
    return pl.pallas_call(
        _tc_d_body,
        grid=(_GRID,),
        in_specs=[pl.BlockSpec((NC, _BLK, ZW), lambda i: (0, i, 0)),
                  pl.BlockSpec((NC, _BLK, 1), lambda i: (0, i, 0)),
                  pl.BlockSpec((_BLK, 2), lambda i: (i, 0)),
                  pl.BlockSpec((2, 1), lambda i: (0, 0))],
        out_specs=pl.BlockSpec((_BLK, 1), lambda i: (i, 0)),
        out_shape=jax.ShapeDtypeStruct((NPAD, 1), jnp.float32),
    )(g2p, deg_part, r2, b2)


def kernel(x, edge_index, W1_init, W1_root, b1, W2_init, W2_root, b2):
    f32 = jnp.float32
    row = edge_index[0].astype(jnp.int32)
    col = edge_index[1].astype(jnp.int32)
    pad_e = jnp.full((EPAD - N_EDGES,), N_NODES, jnp.int32)
    row_r = jnp.concatenate([row, pad_e]).reshape(NW, NCH, CHUNK)
    col_r = jnp.concatenate([col, pad_e]).reshape(NW, NCH, CHUNK)

    x_p = jnp.pad(x.astype(f32), ((0, NPAD - N_NODES), (0, 0)))
    w1_cat = jnp.concatenate(
        [W1_init[0], W1_init[1], W1_root[0], W1_root[1]], axis=1).astype(f32)
    w2_cat = jnp.concatenate(
        [W2_init[0], W2_init[1], W2_root[0], W2_root[1]], axis=1).astype(f32)

    zeros_n = jnp.zeros((NPAD,), f32)
    zeros_nz = jnp.zeros((NPAD, ZW), f32)
    zeros_n32 = jnp.zeros((NPAD, 32), f32)

    p = _tc_a(x_p, w1_cat)
    deg_part = _sc_degree(col_r, zeros_n)
    deg3 = deg_part.reshape(NC, NPAD, 1)
    hs = _tc_b(p, deg3)
    g_part = _sc_pass1(row_r, col_r, hs, zeros_n32)
    z, r2 = _tc_c(g_part, p, deg3, b1.astype(f32), w2_cat)
    g2p = _sc_pass2(row_r, col_r, z, zeros_nz)
    out = _tc_d(g2p, deg3, r2, b2.astype(f32))
    return out[:N_NODES]


# speedup vs baseline: 143.5284x; 1.1792x over previous
import functools

import jax
import jax.numpy as jnp
from jax import lax
from jax.experimental import pallas as pl
from jax.experimental.pallas import tpu as pltpu
from jax.experimental.pallas import tpu_sc as plsc

N_NODES = 50000
N_EDGES = 800000
F_IN = 58

NC = 2
NS = 16
NW = NC * NS
L = 16

CHUNK = 128
NPAD = 50176
RPT = NPAD // NS
EPW = 25088
EPAD = EPW * NW
NCH = EPW // CHUNK
GSZ = 28
ZW = 8
NGRP = NCH // GSZ

_sc_mesh = plsc.VectorSubcoreMesh(core_axis_name="c", subcore_axis_name="s")


def _worker_id():
    return lax.axis_index("c") * NS + lax.axis_index("s")


@functools.partial(
    pl.kernel,
    out_type=jax.ShapeDtypeStruct((NC, NPAD), jnp.float32),
    mesh=_sc_mesh,
    compiler_params=pltpu.CompilerParams(use_tc_tiling_on_sc=False),
    scratch_types=[
        pltpu.VMEM((NCH, CHUNK), jnp.int32),
        pltpu.VMEM((CHUNK,), jnp.float32),
        pltpu.VMEM_SHARED((NPAD,), jnp.float32),
    ],
)
def _sc_degree(col_hbm, zeros_hbm, out_hbm, cidx, ones_v, deg_sh):
    c = lax.axis_index("c")
    s = lax.axis_index("s")
    wid = _worker_id()
    pltpu.sync_copy(zeros_hbm.at[pl.ds(s * RPT, RPT)], deg_sh.at[pl.ds(s * RPT, RPT)])

    def onesb(i, _):
        ones_v[pl.ds(i * L, L)] = jnp.ones((L,), jnp.float32)
        return 0
    lax.fori_loop(0, CHUNK // L, onesb, 0)
    pltpu.sync_copy(col_hbm.at[wid], cidx)
    plsc.subcore_barrier()

    def body(j, _):
        pltpu.sync_copy(ones_v, deg_sh.at[cidx.at[j]], add=True)
        return 0
    lax.fori_loop(0, NCH, body, 0)
    plsc.subcore_barrier()
    pltpu.sync_copy(deg_sh.at[pl.ds(s * RPT, RPT)], out_hbm.at[c, pl.ds(s * RPT, RPT)])


@functools.partial(
    pl.kernel,
    out_type=jax.ShapeDtypeStruct((NC, NPAD, 32), jnp.float32),
    mesh=_sc_mesh,
    compiler_params=pltpu.CompilerParams(use_tc_tiling_on_sc=False),
    scratch_types=[
        pltpu.VMEM((GSZ, CHUNK), jnp.int32),
        pltpu.VMEM((GSZ, CHUNK), jnp.int32),
        pltpu.VMEM((CHUNK, 32), jnp.float32),
        pltpu.VMEM((CHUNK, 32), jnp.float32),
        pltpu.SemaphoreType.DMA,
        pltpu.SemaphoreType.DMA,
        pltpu.VMEM_SHARED((NPAD, 32), jnp.float32),
    ],
)
def _sc_pass1(row_hbm, col_hbm, tab_hbm, zeros_hbm, out_hbm,
              ridx, cidx, data0, data1, sem0, sem1, acc_sh):
    c = lax.axis_index("c")
    s = lax.axis_index("s")
    wid = _worker_id()
    pltpu.sync_copy(zeros_hbm.at[pl.ds(s * RPT, RPT)], acc_sh.at[pl.ds(s * RPT, RPT)])
    plsc.subcore_barrier()

    def grp(gi, _):
        pltpu.sync_copy(row_hbm.at[wid, pl.ds(gi * GSZ, GSZ)], ridx)
        pltpu.sync_copy(col_hbm.at[wid, pl.ds(gi * GSZ, GSZ)], cidx)

        def body(j, _):
            cp0 = pltpu.async_copy(tab_hbm.at[ridx.at[2 * j]], data0, sem0)
            cp1 = pltpu.async_copy(tab_hbm.at[ridx.at[2 * j + 1]], data1, sem1)
            cp0.wait()
            pltpu.sync_copy(data0, acc_sh.at[cidx.at[2 * j]], add=True)
            cp1.wait()
            pltpu.sync_copy(data1, acc_sh.at[cidx.at[2 * j + 1]], add=True)
            return 0
        lax.fori_loop(0, GSZ // 2, body, 0)
        return 0
    lax.fori_loop(0, NGRP, grp, 0)
    plsc.subcore_barrier()
    pltpu.sync_copy(acc_sh.at[pl.ds(s * RPT, RPT)],
                    out_hbm.at[c, pl.ds(s * RPT, RPT), :])


@functools.partial(
    pl.kernel,
    out_type=jax.ShapeDtypeStruct((NC, NPAD, ZW), jnp.float32),
    mesh=_sc_mesh,
    compiler_params=pltpu.CompilerParams(use_tc_tiling_on_sc=False),
    scratch_types=[
        pltpu.VMEM((NCH, CHUNK), jnp.int32),
        pltpu.VMEM((NCH, CHUNK), jnp.int32),
        pltpu.VMEM((CHUNK, ZW), jnp.float32),
        pltpu.VMEM((CHUNK, ZW), jnp.float32),
        pltpu.SemaphoreType.DMA,
        pltpu.SemaphoreType.DMA,
        pltpu.VMEM_SHARED((NPAD, ZW), jnp.float32),
    ],
)
def _sc_pass2(row_hbm, col_hbm, z_hbm, zeros_hbm, out_hbm,
              ridx, cidx, d0, d1, sem0, sem1, g_sh):
    c = lax.axis_index("c")
    s = lax.axis_index("s")
    wid = _worker_id()
    pltpu.sync_copy(zeros_hbm.at[pl.ds(s * RPT, RPT)], g_sh.at[pl.ds(s * RPT, RPT)])
    pltpu.sync_copy(row_hbm.at[wid], ridx)
    pltpu.sync_copy(col_hbm.at[wid], cidx)
    plsc.subcore_barrier()

    def body(j, _):
        cp0 = pltpu.async_copy(z_hbm.at[ridx.at[2 * j]], d0, sem0)
        cp1 = pltpu.async_copy(z_hbm.at[ridx.at[2 * j + 1]], d1, sem1)
        cp0.wait()
        pltpu.sync_copy(d0, g_sh.at[cidx.at[2 * j]], add=True)
        cp1.wait()
        pltpu.sync_copy(d1, g_sh.at[cidx.at[2 * j + 1]], add=True)
        return 0
    lax.fori_loop(0, NCH // 2, body, 0)
    plsc.subcore_barrier()
    pltpu.sync_copy(g_sh.at[pl.ds(s * RPT, RPT)],
                    out_hbm.at[c, pl.ds(s * RPT, RPT), :])


_BLK = RPT
_GRID = NPAD // _BLK


def _tc_a_body(x_ref, w_ref, o_ref):
    o_ref[...] = jnp.dot(x_ref[...], w_ref[...],
                         preferred_element_type=jnp.float32)


def _tc_a(x_p, w_cat):
    return pl.pallas_call(
        _tc_a_body,
        grid=(_GRID,),
        in_specs=[pl.BlockSpec((_BLK, F_IN), lambda i: (i, 0)),
                  pl.BlockSpec((F_IN, 64), lambda i: (0, 0))],
        out_specs=pl.BlockSpec((_BLK, 64), lambda i: (i, 0)),
        out_shape=jax.ShapeDtypeStruct((NPAD, 64), jnp.float32),
    )(x_p, w_cat)


def _dinv_from(degp_ref):
    deg = degp_ref[0, :, 0] + degp_ref[1, :, 0]
    return jnp.where(deg > 0, lax.rsqrt(jnp.maximum(deg, 1e-12)), 0.0)


def _tc_b_body(p_ref, degp_ref, hs_ref):
    dinv = _dinv_from(degp_ref)
    hs_ref[...] = p_ref[:, :32] * dinv[:, None]


def _tc_b(p, deg_part):
    return pl.pallas_call(
        _tc_b_body,
        grid=(_GRID,),
        in_specs=[pl.BlockSpec((_BLK, 64), lambda i: (i, 0)),
                  pl.BlockSpec((NC, _BLK, 1), lambda i: (0, i, 0))],
        out_specs=pl.BlockSpec((_BLK, 32), lambda i: (i, 0)),
        out_shape=jax.ShapeDtypeStruct((NPAD, 32), jnp.float32),
    )(p, deg_part)


def _tc_c_body(gp_ref, p_ref, degp_ref, b1_ref, w2_ref,
               z_ref, r2_ref):
    dinv = _dinv_from(degp_ref)
    g = gp_ref[0] + gp_ref[1]
    agg = g * dinv[:, None]
    pre0 = agg[:, :16] + p_ref[:, 32:48] + b1_ref[0, :][None, :]
    pre1 = agg[:, 16:32] + p_ref[:, 48:64] + b1_ref[1, :][None, :]
    h = jax.nn.relu(0.5 * (jax.nn.relu(pre0) + jax.nn.relu(pre1)))
    z = jnp.dot(h, w2_ref[...], preferred_element_type=jnp.float32)
    zs = z[:, 0:2] * dinv[:, None]
    z_ref[...] = jnp.pad(zs, ((0, 0), (0, ZW - 2)))
    r2_ref[...] = z[:, 2:4]


def _tc_c(g_part, p, deg_part, b1, w2_cat):
    return pl.pallas_call(
        _tc_c_body,
        grid=(_GRID,),
        in_specs=[pl.BlockSpec((NC, _BLK, 32), lambda i: (0, i, 0)),
                  pl.BlockSpec((_BLK, 64), lambda i: (i, 0)),
                  pl.BlockSpec((NC, _BLK, 1), lambda i: (0, i, 0)),
                  pl.BlockSpec((2, 16), lambda i: (0, 0)),
                  pl.BlockSpec((16, 4), lambda i: (0, 0))],
        out_specs=[pl.BlockSpec((_BLK, ZW), lambda i: (i, 0)),
                   pl.BlockSpec((_BLK, 2), lambda i: (i, 0))],
        out_shape=[jax.ShapeDtypeStruct((NPAD, ZW), jnp.float32),
                   jax.ShapeDtypeStruct((NPAD, 2), jnp.float32)],
    )(g_part, p, deg_part, b1, w2_cat)


def _tc_d_body(g_ref, degp_ref, r2_ref, b2_ref, o_ref):
    dinv = _dinv_from(degp_ref)
    gsum = g_ref[0] + g_ref[1]
    o0 = jax.nn.relu(dinv * gsum[:, 0] + r2_ref[:, 0] + b2_ref[0, 0])
    o1 = jax.nn.relu(dinv * gsum[:, 1] + r2_ref[:, 1] + b2_ref[1, 0])
    o_ref[...] = (0.5 * (o0 + o1))[:, None]


def _tc_d(g2p, deg_part, r2, b2):
    return pl.pallas_call(
        _tc_d_body,
        grid=(_GRID,),
        in_specs=[pl.BlockSpec((NC, _BLK, ZW), lambda i: (0, i, 0)),
                  pl.BlockSpec((NC, _BLK, 1), lambda i: (0, i, 0)),
                  pl.BlockSpec((_BLK, 2), lambda i: (i, 0)),
                  pl.BlockSpec((2, 1), lambda i: (0, 0))],
        out_specs=pl.BlockSpec((_BLK, 1), lambda i: (i, 0)),
        out_shape=jax.ShapeDtypeStruct((NPAD, 1), jnp.float32),
    )(g2p, deg_part, r2, b2)


def kernel(x, edge_index, W1_init, W1_root, b1, W2_init, W2_root, b2):
    f32 = jnp.float32
    row = edge_index[0].astype(jnp.int32)
    col = edge_index[1].astype(jnp.int32)
    pad_e = jnp.full((EPAD - N_EDGES,), N_NODES, jnp.int32)
    row_r = jnp.concatenate([row, pad_e]).reshape(NW, NCH, CHUNK)
    col_r = jnp.concatenate([col, pad_e]).reshape(NW, NCH, CHUNK)

    x_p = jnp.pad(x.astype(f32), ((0, NPAD - N_NODES), (0, 0)))
    w1_cat = jnp.concatenate(
        [W1_init[0], W1_init[1], W1_root[0], W1_root[1]], axis=1).astype(f32)
    w2_cat = jnp.concatenate(
        [W2_init[0], W2_init[1], W2_root[0], W2_root[1]], axis=1).astype(f32)

    zeros_n = jnp.zeros((NPAD,), f32)
    zeros_nz = jnp.zeros((NPAD, ZW), f32)
    zeros_n32 = jnp.zeros((NPAD, 32), f32)

    p = _tc_a(x_p, w1_cat)
    deg_part = _sc_degree(col_r, zeros_n)
    deg3 = deg_part.reshape(NC, NPAD, 1)
    hs = _tc_b(p, deg3)
    g_part = _sc_pass1(row_r, col_r, hs, zeros_n32)
    z, r2 = _tc_c(g_part, p, deg3, b1.astype(f32), w2_cat)
    g2p = _sc_pass2(row_r, col_r, z, zeros_nz)
    out = _tc_d(g2p, deg3, r2, b2.astype(f32))
    return out[:N_NODES]


# speedup vs baseline: 154.9973x; 1.0799x over previous
import functools

import jax
import jax.numpy as jnp
from jax import lax
from jax.experimental import pallas as pl
from jax.experimental.pallas import tpu as pltpu
from jax.experimental.pallas import tpu_sc as plsc

N_NODES = 50000
N_EDGES = 800000
F_IN = 58

NC = 2
NS = 16
NW = NC * NS
L = 16

CHUNK = 128
NPAD = 50176
RPT = NPAD // NS
EPW = 25088
EPAD = EPW * NW
NCH = EPW // CHUNK
GSZ = 28
ZW = 8
NGRP = NCH // GSZ

_sc_mesh = plsc.VectorSubcoreMesh(core_axis_name="c", subcore_axis_name="s")


def _worker_id():
    return lax.axis_index("c") * NS + lax.axis_index("s")


@functools.partial(
    pl.kernel,
    out_type=jax.ShapeDtypeStruct((NC, NPAD), jnp.float32),
    mesh=_sc_mesh,
    compiler_params=pltpu.CompilerParams(use_tc_tiling_on_sc=False),
    scratch_types=[
        pltpu.VMEM((NCH, CHUNK), jnp.int32),
        pltpu.VMEM((CHUNK,), jnp.float32),
        pltpu.VMEM_SHARED((NPAD,), jnp.float32),
    ],
)
def _sc_degree(col_hbm, zeros_hbm, out_hbm, cidx, ones_v, deg_sh):
    c = lax.axis_index("c")
    s = lax.axis_index("s")
    wid = _worker_id()
    pltpu.sync_copy(zeros_hbm.at[pl.ds(s * RPT, RPT)], deg_sh.at[pl.ds(s * RPT, RPT)])

    def onesb(i, _):
        ones_v[pl.ds(i * L, L)] = jnp.ones((L,), jnp.float32)
        return 0
    lax.fori_loop(0, CHUNK // L, onesb, 0)
    pltpu.sync_copy(col_hbm.at[wid], cidx)
    plsc.subcore_barrier()

    def body(j, _):
        pltpu.sync_copy(ones_v, deg_sh.at[cidx.at[j]], add=True)
        return 0
    lax.fori_loop(0, NCH, body, 0)
    plsc.subcore_barrier()
    pltpu.sync_copy(deg_sh.at[pl.ds(s * RPT, RPT)], out_hbm.at[c, pl.ds(s * RPT, RPT)])


@functools.partial(
    pl.kernel,
    out_type=jax.ShapeDtypeStruct((NC, NPAD, 32), jnp.float32),
    mesh=_sc_mesh,
    compiler_params=pltpu.CompilerParams(use_tc_tiling_on_sc=False),
    scratch_types=[
        pltpu.VMEM((GSZ, CHUNK), jnp.int32),
        pltpu.VMEM((GSZ, CHUNK), jnp.int32),
        pltpu.VMEM((CHUNK, 32), jnp.float32),
        pltpu.VMEM((CHUNK, 32), jnp.float32),
        pltpu.VMEM((CHUNK, 32), jnp.float32),
        pltpu.VMEM((CHUNK, 32), jnp.float32),
        pltpu.SemaphoreType.DMA,
        pltpu.SemaphoreType.DMA,
        pltpu.SemaphoreType.DMA,
        pltpu.SemaphoreType.DMA,
        pltpu.VMEM_SHARED((NPAD, 32), jnp.float32),
    ],
)
def _sc_pass1(row_hbm, col_hbm, tab_hbm, zeros_hbm, out_hbm,
              ridx, cidx, data0, data1, data2, data3,
              sem0, sem1, sem2, sem3, acc_sh):
    c = lax.axis_index("c")
    s = lax.axis_index("s")
    wid = _worker_id()
    pltpu.sync_copy(zeros_hbm.at[pl.ds(s * RPT, RPT)], acc_sh.at[pl.ds(s * RPT, RPT)])
    plsc.subcore_barrier()

    def grp(gi, _):
        pltpu.sync_copy(row_hbm.at[wid, pl.ds(gi * GSZ, GSZ)], ridx)
        pltpu.sync_copy(col_hbm.at[wid, pl.ds(gi * GSZ, GSZ)], cidx)

        def body(j, _):
            cp0 = pltpu.async_copy(tab_hbm.at[ridx.at[4 * j]], data0, sem0)
            cp1 = pltpu.async_copy(tab_hbm.at[ridx.at[4 * j + 1]], data1, sem1)
            cp2 = pltpu.async_copy(tab_hbm.at[ridx.at[4 * j + 2]], data2, sem2)
            cp3 = pltpu.async_copy(tab_hbm.at[ridx.at[4 * j + 3]], data3, sem3)
            cp0.wait()
            pltpu.sync_copy(data0, acc_sh.at[cidx.at[4 * j]], add=True)
            cp1.wait()
            pltpu.sync_copy(data1, acc_sh.at[cidx.at[4 * j + 1]], add=True)
            cp2.wait()
            pltpu.sync_copy(data2, acc_sh.at[cidx.at[4 * j + 2]], add=True)
            cp3.wait()
            pltpu.sync_copy(data3, acc_sh.at[cidx.at[4 * j + 3]], add=True)
            return 0
        lax.fori_loop(0, GSZ // 4, body, 0)
        return 0
    lax.fori_loop(0, NGRP, grp, 0)
    plsc.subcore_barrier()
    pltpu.sync_copy(acc_sh.at[pl.ds(s * RPT, RPT)],
                    out_hbm.at[c, pl.ds(s * RPT, RPT), :])


@functools.partial(
    pl.kernel,
    out_type=jax.ShapeDtypeStruct((NC, NPAD, ZW), jnp.float32),
    mesh=_sc_mesh,
    compiler_params=pltpu.CompilerParams(use_tc_tiling_on_sc=False),
    scratch_types=[
        pltpu.VMEM((NCH, CHUNK), jnp.int32),
        pltpu.VMEM((NCH, CHUNK), jnp.int32),
        pltpu.VMEM((CHUNK, ZW), jnp.float32),
        pltpu.VMEM((CHUNK, ZW), jnp.float32),
        pltpu.VMEM((CHUNK, ZW), jnp.float32),
        pltpu.VMEM((CHUNK, ZW), jnp.float32),
        pltpu.SemaphoreType.DMA,
        pltpu.SemaphoreType.DMA,
        pltpu.SemaphoreType.DMA,
        pltpu.SemaphoreType.DMA,
        pltpu.VMEM_SHARED((NPAD, ZW), jnp.float32),
    ],
)
def _sc_pass2(row_hbm, col_hbm, z_hbm, zeros_hbm, out_hbm,
              ridx, cidx, d0, d1, d2, d3, sem0, sem1, sem2, sem3, g_sh):
    c = lax.axis_index("c")
    s = lax.axis_index("s")
    wid = _worker_id()
    pltpu.sync_copy(zeros_hbm.at[pl.ds(s * RPT, RPT)], g_sh.at[pl.ds(s * RPT, RPT)])
    pltpu.sync_copy(row_hbm.at[wid], ridx)
    pltpu.sync_copy(col_hbm.at[wid], cidx)
    plsc.subcore_barrier()

    def body(j, _):
        cp0 = pltpu.async_copy(z_hbm.at[ridx.at[4 * j]], d0, sem0)
        cp1 = pltpu.async_copy(z_hbm.at[ridx.at[4 * j + 1]], d1, sem1)
        cp2 = pltpu.async_copy(z_hbm.at[ridx.at[4 * j + 2]], d2, sem2)
        cp3 = pltpu.async_copy(z_hbm.at[ridx.at[4 * j + 3]], d3, sem3)
        cp0.wait()
        pltpu.sync_copy(d0, g_sh.at[cidx.at[4 * j]], add=True)
        cp1.wait()
        pltpu.sync_copy(d1, g_sh.at[cidx.at[4 * j + 1]], add=True)
        cp2.wait()
        pltpu.sync_copy(d2, g_sh.at[cidx.at[4 * j + 2]], add=True)
        cp3.wait()
        pltpu.sync_copy(d3, g_sh.at[cidx.at[4 * j + 3]], add=True)
        return 0
    lax.fori_loop(0, NCH // 4, body, 0)
    plsc.subcore_barrier()
    pltpu.sync_copy(g_sh.at[pl.ds(s * RPT, RPT)],
                    out_hbm.at[c, pl.ds(s * RPT, RPT), :])


_BLK = RPT
_GRID = NPAD // _BLK


def _tc_a_body(x_ref, w_ref, o_ref):
    o_ref[...] = jnp.dot(x_ref[...], w_ref[...],
                         preferred_element_type=jnp.float32)


def _tc_a(x_p, w_cat):
    return pl.pallas_call(
        _tc_a_body,
        grid=(_GRID,),
        in_specs=[pl.BlockSpec((_BLK, F_IN), lambda i: (i, 0)),
                  pl.BlockSpec((F_IN, 64), lambda i: (0, 0))],
        out_specs=pl.BlockSpec((_BLK, 64), lambda i: (i, 0)),
        out_shape=jax.ShapeDtypeStruct((NPAD, 64), jnp.float32),
    )(x_p, w_cat)


def _dinv_from(degp_ref):
    deg = degp_ref[0, :, 0] + degp_ref[1, :, 0]
    return jnp.where(deg > 0, lax.rsqrt(jnp.maximum(deg, 1e-12)), 0.0)


def _tc_b_body(p_ref, degp_ref, hs_ref):
    dinv = _dinv_from(degp_ref)
    hs_ref[...] = p_ref[:, :32] * dinv[:, None]


def _tc_b(p, deg_part):
    return pl.pallas_call(
        _tc_b_body,
        grid=(_GRID,),
        in_specs=[pl.BlockSpec((_BLK, 64), lambda i: (i, 0)),
                  pl.BlockSpec((NC, _BLK, 1), lambda i: (0, i, 0))],
        out_specs=pl.BlockSpec((_BLK, 32), lambda i: (i, 0)),
        out_shape=jax.ShapeDtypeStruct((NPAD, 32), jnp.float32),
    )(p, deg_part)


def _tc_c_body(gp_ref, p_ref, degp_ref, b1_ref, w2_ref,
               z_ref, r2_ref):
    dinv = _dinv_from(degp_ref)
    g = gp_ref[0] + gp_ref[1]
    agg = g * dinv[:, None]
    pre0 = agg[:, :16] + p_ref[:, 32:48] + b1_ref[0, :][None, :]
    pre1 = agg[:, 16:32] + p_ref[:, 48:64] + b1_ref[1, :][None, :]
    h = jax.nn.relu(0.5 * (jax.nn.relu(pre0) + jax.nn.relu(pre1)))
    z = jnp.dot(h, w2_ref[...], preferred_element_type=jnp.float32)
    zs = z[:, 0:2] * dinv[:, None]
    z_ref[...] = jnp.pad(zs, ((0, 0), (0, ZW - 2)))
    r2_ref[...] = z[:, 2:4]


def _tc_c(g_part, p, deg_part, b1, w2_cat):
    return pl.pallas_call(
        _tc_c_body,
        grid=(_GRID,),
        in_specs=[pl.BlockSpec((NC, _BLK, 32), lambda i: (0, i, 0)),
                  pl.BlockSpec((_BLK, 64), lambda i: (i, 0)),
                  pl.BlockSpec((NC, _BLK, 1), lambda i: (0, i, 0)),
                  pl.BlockSpec((2, 16), lambda i: (0, 0)),
                  pl.BlockSpec((16, 4), lambda i: (0, 0))],
        out_specs=[pl.BlockSpec((_BLK, ZW), lambda i: (i, 0)),
                   pl.BlockSpec((_BLK, 2), lambda i: (i, 0))],
        out_shape=[jax.ShapeDtypeStruct((NPAD, ZW), jnp.float32),
                   jax.ShapeDtypeStruct((NPAD, 2), jnp.float32)],
    )(g_part, p, deg_part, b1, w2_cat)


def _tc_d_body(g_ref, degp_ref, r2_ref, b2_ref, o_ref):
    dinv = _dinv_from(degp_ref)
    gsum = g_ref[0] + g_ref[1]
    o0 = jax.nn.relu(dinv * gsum[:, 0] + r2_ref[:, 0] + b2_ref[0, 0])
    o1 = jax.nn.relu(dinv * gsum[:, 1] + r2_ref[:, 1] + b2_ref[1, 0])
    o_ref[...] = (0.5 * (o0 + o1))[:, None]


def _tc_d(g2p, deg_part, r2, b2):
    return pl.pallas_call(
        _tc_d_body,
        grid=(_GRID,),
        in_specs=[pl.BlockSpec((NC, _BLK, ZW), lambda i: (0, i, 0)),
                  pl.BlockSpec((NC, _BLK, 1), lambda i: (0, i, 0)),
                  pl.BlockSpec((_BLK, 2), lambda i: (i, 0)),
                  pl.BlockSpec((2, 1), lambda i: (0, 0))],
        out_specs=pl.BlockSpec((_BLK, 1), lambda i: (i, 0)),
        out_shape=jax.ShapeDtypeStruct((NPAD, 1), jnp.float32),
    )(g2p, deg_part, r2, b2)


def kernel(x, edge_index, W1_init, W1_root, b1, W2_init, W2_root, b2):
    f32 = jnp.float32
    row = edge_index[0].astype(jnp.int32)
    col = edge_index[1].astype(jnp.int32)
    pad_e = jnp.full((EPAD - N_EDGES,), N_NODES, jnp.int32)
    row_r = jnp.concatenate([row, pad_e]).reshape(NW, NCH, CHUNK)
    col_r = jnp.concatenate([col, pad_e]).reshape(NW, NCH, CHUNK)

    x_p = jnp.pad(x.astype(f32), ((0, NPAD - N_NODES), (0, 0)))
    w1_cat = jnp.concatenate(
        [W1_init[0], W1_init[1], W1_root[0], W1_root[1]], axis=1).astype(f32)
    w2_cat = jnp.concatenate(
        [W2_init[0], W2_init[1], W2_root[0], W2_root[1]], axis=1).astype(f32)

    zeros_n = jnp.zeros((NPAD,), f32)
    zeros_nz = jnp.zeros((NPAD, ZW), f32)
    zeros_n32 = jnp.zeros((NPAD, 32), f32)

    p = _tc_a(x_p, w1_cat)
    deg_part = _sc_degree(col_r, zeros_n)
    deg3 = deg_part.reshape(NC, NPAD, 1)
    hs = _tc_b(p, deg3)
    g_part = _sc_pass1(row_r, col_r, hs, zeros_n32)
    z, r2 = _tc_c(g_part, p, deg3, b1.astype(f32), w2_cat)
    g2p = _sc_pass2(row_r, col_r, z, zeros_nz)
    out = _tc_d(g2p, deg3, r2, b2.astype(f32))
    return out[:N_NODES]
